# baseline (speedup 1.0000x reference)
import functools

import jax
import jax.numpy as jnp
from jax import lax
from jax.experimental import pallas as pl
from jax.experimental.pallas import tpu as pltpu
from jax.experimental.pallas import tpu_sc as plsc

D_MODEL = 64
R_CHUNK = 4
NBUF = 2
SPLITS = ((0, 128), (128, 72))


def _gather_impl(x, emb_weight):
    B, L = x.shape
    info = plsc.get_sparse_core_info()
    NC, NS = info.num_cores, info.num_subcores
    NW = NC * NS
    rows_per_w = B // NW
    n_chunks = rows_per_w // R_CHUNK
    assert n_chunks % NBUF == 0
    mesh = plsc.VectorSubcoreMesh(core_axis_name="c", subcore_axis_name="s")

    @functools.partial(
        pl.kernel,
        mesh=mesh,
        out_type=jax.ShapeDtypeStruct((B, L, D_MODEL), jnp.float32),
        scratch_types=[
            pltpu.VMEM((NBUF, R_CHUNK, L), jnp.int32),
            pltpu.VMEM((NBUF, R_CHUNK, L, D_MODEL), jnp.float32),
            pltpu.SemaphoreType.DMA,
            pltpu.SemaphoreType.DMA,
            pltpu.SemaphoreType.DMA,
        ],
        compiler_params=pltpu.CompilerParams(use_tc_tiling_on_sc=False),
    )
    def k(x_hbm, tab_hbm, out_hbm, idx_v, rows_v, isem, gsem, osem):
        wid = lax.axis_index("s") * NC + lax.axis_index("c")
        base = wid * rows_per_w

        def issue_idx(i, b):
            pltpu.async_copy(
                x_hbm.at[pl.ds(base + i * R_CHUNK, R_CHUNK)],
                idx_v.at[b], isem)

        def wait_idx(b):
            pltpu.make_async_copy(
                x_hbm.at[pl.ds(0, R_CHUNK)], idx_v.at[b], isem).wait()

        def issue_gathers(b):
            for r in range(R_CHUNK):
                for (o, ln) in SPLITS:
                    pltpu.async_copy(
                        tab_hbm.at[idx_v.at[b, r, pl.ds(o, ln)]],
                        rows_v.at[b, r, pl.ds(o, ln)],
                        gsem,
                    )

        def wait_gathers(b):
            for r in range(R_CHUNK):
                for (o, ln) in SPLITS:
                    pltpu.make_async_copy(
                        out_hbm.at[0, pl.ds(o, ln)],
                        rows_v.at[b, r, pl.ds(o, ln)],
                        gsem,
                    ).wait()

        def issue_out(i, b):
            pltpu.async_copy(
                rows_v.at[b],
                out_hbm.at[pl.ds(base + i * R_CHUNK, R_CHUNK)], osem)

        def wait_out(b):
            pltpu.make_async_copy(
                rows_v.at[b],
                out_hbm.at[pl.ds(0, R_CHUNK)], osem).wait()

        for b in range(NBUF):
            issue_idx(b, b)

        def pair_body(t, carry):
            for b in range(NBUF):
                i = t * NBUF + b

                @pl.when(i >= NBUF)
                def _():
                    wait_out(b)

                wait_idx(b)
                issue_gathers(b)
                wait_gathers(b)
                issue_out(i, b)

                @pl.when(i + NBUF < n_chunks)
                def _():
                    issue_idx(i + NBUF, b)
            return carry

        lax.fori_loop(0, n_chunks // NBUF, pair_body, 0)

        for b in range(NBUF):
            wait_out(b)

    return k(x, emb_weight)


TAB_CHUNK = 320


def _pack_pairs(emb_weight):
    V = emb_weight.shape[0]
    info = plsc.get_sparse_core_info()
    NC, NS = info.num_cores, info.num_subcores
    NW = NC * NS
    n_chunks = V // TAB_CHUNK
    rounds = (n_chunks + NW - 1) // NW
    assert rounds % NBUF == 0
    mesh = plsc.VectorSubcoreMesh(core_axis_name="c", subcore_axis_name="s")

    @functools.partial(
        pl.kernel,
        mesh=mesh,
        out_type=jax.ShapeDtypeStruct((V // 2, 128), jnp.float32),
        scratch_types=[
            pltpu.VMEM((NBUF, TAB_CHUNK, D_MODEL), jnp.float32),
            pltpu.VMEM((NBUF, TAB_CHUNK // 2, 2 * D_MODEL), jnp.float32),
            pltpu.SemaphoreType.DMA,
            pltpu.SemaphoreType.DMA,
        ],
        compiler_params=pltpu.CompilerParams(use_tc_tiling_on_sc=True, needs_layout_passes=False),
    )
    def k(tab_hbm, tabp_hbm, v64, v128, rsem, wsem):
        wid = lax.axis_index("s") * NC + lax.axis_index("c")

        def chunk_of(j):
            return wid + j * NW

        def issue_read(j, b):
            c = chunk_of(j)

            @pl.when(c < n_chunks)
            def _():
                pltpu.async_copy(
                    tab_hbm.at[pl.ds(pl.multiple_of(c * TAB_CHUNK, 8), TAB_CHUNK)],
                    v64.at[b], rsem)

        def wait_read(j, b):
            @pl.when(chunk_of(j) < n_chunks)
            def _():
                pltpu.make_async_copy(
                    tab_hbm.at[pl.ds(0, TAB_CHUNK)], v64.at[b], rsem).wait()

        def repack(j, b):
            @pl.when(chunk_of(j) < n_chunks)
            def _():
                lane = lax.iota(jnp.int32, 16)

                def body(kk, carry):
                    krow = jnp.full((16,), kk, jnp.int32)
                    for half in range(2):
                        rows = jnp.full((16,), 2 * kk + half, jnp.int32)
                        for j4 in range(4):
                            cols = lane + 16 * j4
                            val = plsc.load_gather(v64.at[b], [rows, cols])
                            plsc.store_scatter(
                                v128.at[b], [krow, cols + 64 * half], val)
                    return carry

                lax.fori_loop(0, TAB_CHUNK // 2, body, 0)

        def issue_write(j, b):
            c = chunk_of(j)

            @pl.when(c < n_chunks)
            def _():
                pltpu.async_copy(
                    v128.at[b],
                    tabp_hbm.at[pl.ds(pl.multiple_of(c * (TAB_CHUNK // 2), 8), TAB_CHUNK // 2)],
                    wsem)

        def wait_write(j, b):
            @pl.when(chunk_of(j) < n_chunks)
            def _():
                pltpu.make_async_copy(
                    v128.at[b],
                    tabp_hbm.at[pl.ds(0, TAB_CHUNK // 2)], wsem).wait()

        for b in range(NBUF):
            issue_read(b, b)

        def pair_body(t, carry):
            for b in range(NBUF):
                j = t * NBUF + b

                @pl.when(j >= NBUF)
                def _():
                    wait_write(j - NBUF, b)

                wait_read(j, b)
                repack(j, b)
                issue_write(j, b)

                @pl.when(j + NBUF < rounds)
                def _():
                    issue_read(j + NBUF, b)
            return carry

        lax.fori_loop(0, rounds // NBUF, pair_body, 0)

        for b in range(NBUF):
            wait_write(rounds - NBUF + b, b)

    return k(emb_weight)


def kernel(x, emb_weight):
    tabp = _pack_pairs(emb_weight)
    tab_lin = tabp.reshape(emb_weight.shape[0], D_MODEL)
    return _gather_impl(x.astype(jnp.int32), tab_lin)


# speedup vs baseline: 1.1954x; 1.1954x over previous
import functools

import jax
import jax.numpy as jnp
from jax import lax
from jax.experimental import pallas as pl
from jax.experimental.pallas import tpu as pltpu
from jax.experimental.pallas import tpu_sc as plsc

D_MODEL = 64
R_CHUNK = 4
NBUF = 2
SPLITS = ((0, 128), (128, 72))


def _gather_impl(x, emb_weight):
    B, L = x.shape
    info = plsc.get_sparse_core_info()
    NC, NS = info.num_cores, info.num_subcores
    NW = NC * NS
    rows_per_w = B // NW
    n_chunks = rows_per_w // R_CHUNK
    assert n_chunks % NBUF == 0
    mesh = plsc.VectorSubcoreMesh(core_axis_name="c", subcore_axis_name="s")

    @functools.partial(
        pl.kernel,
        mesh=mesh,
        out_type=jax.ShapeDtypeStruct((B, L, D_MODEL), jnp.float32),
        scratch_types=[
            pltpu.VMEM((NBUF, R_CHUNK, L), jnp.int32),
            pltpu.VMEM((NBUF, R_CHUNK, L, D_MODEL), jnp.float32),
            pltpu.SemaphoreType.DMA,
            pltpu.SemaphoreType.DMA,
            pltpu.SemaphoreType.DMA,
        ],
        compiler_params=pltpu.CompilerParams(use_tc_tiling_on_sc=False),
    )
    def k(x_hbm, tab_hbm, out_hbm, idx_v, rows_v, isem, gsem, osem):
        wid = lax.axis_index("s") * NC + lax.axis_index("c")
        base = wid * rows_per_w

        def issue_idx(i, b):
            pltpu.async_copy(
                x_hbm.at[pl.ds(base + i * R_CHUNK, R_CHUNK)],
                idx_v.at[b], isem)

        def wait_idx(b):
            pltpu.make_async_copy(
                x_hbm.at[pl.ds(0, R_CHUNK)], idx_v.at[b], isem).wait()

        def issue_gathers(b):
            for r in range(R_CHUNK):
                for (o, ln) in SPLITS:
                    pltpu.async_copy(
                        tab_hbm.at[idx_v.at[b, r, pl.ds(o, ln)]],
                        rows_v.at[b, r, pl.ds(o, ln)],
                        gsem,
                    )

        def wait_gathers(b):
            for r in range(R_CHUNK):
                for (o, ln) in SPLITS:
                    pltpu.make_async_copy(
                        out_hbm.at[0, pl.ds(o, ln)],
                        rows_v.at[b, r, pl.ds(o, ln)],
                        gsem,
                    ).wait()

        def issue_out(i, b):
            pltpu.async_copy(
                rows_v.at[b],
                out_hbm.at[pl.ds(base + i * R_CHUNK, R_CHUNK)], osem)

        def wait_out(b):
            pltpu.make_async_copy(
                rows_v.at[b],
                out_hbm.at[pl.ds(0, R_CHUNK)], osem).wait()

        for b in range(NBUF):
            issue_idx(b, b)

        def pair_body(t, carry):
            for b in range(NBUF):
                i = t * NBUF + b

                @pl.when(i >= NBUF)
                def _():
                    wait_out(b)

                wait_idx(b)
                issue_gathers(b)
                wait_gathers(b)
                issue_out(i, b)

                @pl.when(i + NBUF < n_chunks)
                def _():
                    issue_idx(i + NBUF, b)
            return carry

        lax.fori_loop(0, n_chunks // NBUF, pair_body, 0)

        for b in range(NBUF):
            wait_out(b)

    return k(x, emb_weight)


def kernel(x, emb_weight):
    return _gather_impl(x.astype(jnp.int32), emb_weight)
